# baseline (speedup 1.0000x reference)
import functools
import jax
import jax.numpy as jnp
from jax import lax
from jax.experimental import pallas as pl
from jax.experimental.pallas import tpu as pltpu
from jax.experimental.pallas import tpu_sc as plsc

N = 10000
E = 160000
D = 256
HALF = 128

NC = 2
NS = 16
CHUNK = 128
NCHUNK = 80
EPT = CHUNK * NCHUNK
E_PAD = EPT * NS
AGG_ROWS = 10240
ROWS_PT = AGG_ROWS // NS
TRASH = N

HR = 80
HPT = 8
HTILES = HR // HPT

_MESH = plsc.VectorSubcoreMesh(core_axis_name="c", subcore_axis_name="s")
_PARAMS = pltpu.CompilerParams(needs_layout_passes=False)


@functools.partial(
    pl.kernel, mesh=_MESH,
    out_type=jax.ShapeDtypeStruct((HR, HALF), jnp.float32),
    scratch_types=[
        pltpu.VMEM((NCHUNK, CHUNK), jnp.int32),
        pltpu.VMEM((HR, HALF), jnp.float32),
        pltpu.VMEM_SHARED((NS, HR, HALF), jnp.float32),
        pltpu.VMEM((HPT, HALF), jnp.float32),
        pltpu.VMEM((HPT, HALF), jnp.float32),
    ],
    compiler_params=_PARAMS)
def _sc_deg(dst3_hbm, deg_hbm, didx_v, hist_v, hist_sp, tmp_v, acc_v):
    c = lax.axis_index("c")
    s = lax.axis_index("s")
    zero16 = jnp.zeros((16,), jnp.float32)
    ones16 = jnp.ones((16,), jnp.float32)

    pltpu.sync_copy(dst3_hbm.at[s], didx_v)

    def hzbody(i, carry):
        for kk in range(HALF // 16):
            hist_v[i, pl.ds(kk * 16, 16)] = zero16
        return carry
    lax.fori_loop(0, HR, hzbody, 0)

    def hbody(j, carry):
        for kk in range(CHUNK // 16):
            d16 = didx_v[j, pl.ds(kk * 16, 16)]
            plsc.addupdate_scatter(
                hist_v,
                [lax.shift_right_logical(d16, 7),
                 lax.bitwise_and(d16, 127)],
                ones16)
        return carry
    lax.fori_loop(0, NCHUNK, hbody, 0)

    @pl.when(c == 0)
    def _():
        pltpu.sync_copy(hist_v, hist_sp.at[s])
    plsc.subcore_barrier()

    @pl.when(jnp.logical_and(c == 0, s < HTILES))
    def _():
        def azbody(i, carry):
            acc_v[i // 8, pl.ds((i % 8) * 16, 16)] = zero16
            return carry
        lax.fori_loop(0, HPT * (HALF // 16), azbody, 0)
        for t in range(NS):
            pltpu.sync_copy(
                hist_sp.at[t, pl.ds(pl.multiple_of(s * HPT, 8), HPT)],
                tmp_v)

            def abody(i, carry2):
                rr = i // 8
                sl = pl.ds((i % 8) * 16, 16)
                acc_v[rr, sl] = acc_v[rr, sl] + tmp_v[rr, sl]
                return carry2
            lax.fori_loop(0, HPT * (HALF // 16), abody, 0)
        pltpu.sync_copy(
            acc_v, deg_hbm.at[pl.ds(pl.multiple_of(s * HPT, 8), HPT)])


@functools.partial(
    pl.kernel, mesh=_MESH,
    out_type=jax.ShapeDtypeStruct((NC, AGG_ROWS, HALF), jnp.float32),
    scratch_types=[
        pltpu.VMEM((NCHUNK, CHUNK), jnp.int32),
        pltpu.VMEM((NCHUNK, CHUNK), jnp.int32),
        pltpu.VMEM((CHUNK, HALF), jnp.float32),
        pltpu.VMEM_SHARED((AGG_ROWS, HALF), jnp.float32),
        pltpu.SemaphoreType.DMA,
    ],
    compiler_params=_PARAMS)
def _sc_segsum(x2_hbm, gsrc_hbm, dst3_hbm, out_hbm,
               idx_v, didx_v, rows_v, agg_sp, sem):
    c = lax.axis_index("c")
    s = lax.axis_index("s")
    zero16 = jnp.zeros((16,), jnp.float32)

    pltpu.sync_copy(gsrc_hbm.at[c, s], idx_v)
    pltpu.sync_copy(dst3_hbm.at[s], didx_v)

    def zbody(i, carry):
        for kk in range(HALF // 16):
            rows_v[i, pl.ds(kk * 16, 16)] = zero16
        return carry
    lax.fori_loop(0, CHUNK, zbody, 0)
    base = s * ROWS_PT
    for q in range(ROWS_PT // CHUNK):
        pltpu.sync_copy(rows_v, agg_sp.at[pl.ds(base + q * CHUNK, CHUNK)])
    _zrem = ROWS_PT % CHUNK
    if _zrem:
        pltpu.sync_copy(
            rows_v.at[pl.ds(0, _zrem)],
            agg_sp.at[pl.ds(base + (ROWS_PT // CHUNK) * CHUNK, _zrem)])
    plsc.subcore_barrier()

    def body(j, carry):
        pltpu.async_copy(x2_hbm.at[idx_v.at[j]], rows_v, sem).wait()
        pltpu.sync_copy(rows_v, agg_sp.at[didx_v.at[j]], add=True)
        return carry
    lax.fori_loop(0, NCHUNK, body, 0)
    plsc.subcore_barrier()

    pltpu.sync_copy(agg_sp.at[pl.ds(base, ROWS_PT)],
                    out_hbm.at[c, pl.ds(base, ROWS_PT)])


_BLK = 1000


def _tca_body(x_ref, ws_ref, b_ref, p_ref):
    p_ref[...] = jnp.dot(x_ref[...], ws_ref[...],
                         preferred_element_type=jnp.float32) + b_ref[...]


def _tca(x, w_self, b):
    return pl.pallas_call(
        _tca_body,
        grid=(N // _BLK,),
        in_specs=[
            pl.BlockSpec((_BLK, D), lambda i: (i, 0)),
            pl.BlockSpec((D, D), lambda i: (0, 0)),
            pl.BlockSpec((1, D), lambda i: (0, 0)),
        ],
        out_specs=pl.BlockSpec((_BLK, D), lambda i: (i, 0)),
        out_shape=jax.ShapeDtypeStruct((N, D), jnp.float32),
    )(x, w_self, b.reshape(1, D))


def _tcb_body(x_ref, agg_ref, deg_ref, ws_ref, wn_ref, b_ref, y_ref, y2_ref):
    r = 1.0 / jnp.maximum(deg_ref[...], 1.0)
    lo = agg_ref[0] * r
    hi = agg_ref[1] * r
    acc = jnp.dot(x_ref[...], ws_ref[...], preferred_element_type=jnp.float32)
    acc += jnp.dot(lo, wn_ref[0:HALF, :], preferred_element_type=jnp.float32)
    acc += jnp.dot(hi, wn_ref[HALF:, :], preferred_element_type=jnp.float32)
    y = jnp.maximum(acc + b_ref[...], 0.0)
    y_ref[...] = y
    y2_ref[0] = y[:, :HALF]
    y2_ref[1] = y[:, HALF:]


def _tcb(x, agg2, degc, w_self, w_neigh, b):
    return pl.pallas_call(
        _tcb_body,
        grid=(N // _BLK,),
        in_specs=[
            pl.BlockSpec((_BLK, D), lambda i: (i, 0)),
            pl.BlockSpec((NC, _BLK, HALF), lambda i: (0, i, 0)),
            pl.BlockSpec((_BLK, 1), lambda i: (i, 0)),
            pl.BlockSpec((D, D), lambda i: (0, 0)),
            pl.BlockSpec((D, D), lambda i: (0, 0)),
            pl.BlockSpec((1, D), lambda i: (0, 0)),
        ],
        out_specs=[
            pl.BlockSpec((_BLK, D), lambda i: (i, 0)),
            pl.BlockSpec((NC, _BLK, HALF), lambda i: (0, i, 0)),
        ],
        out_shape=[
            jax.ShapeDtypeStruct((N, D), jnp.float32),
            jax.ShapeDtypeStruct((NC, N, HALF), jnp.float32),
        ],
    )(x, agg2, degc, w_self, w_neigh, b.reshape(1, D))


def kernel(h, edge_index, W_self0, W_neigh0, b0, W_self1, W_neigh1, b1):
    src = edge_index[0]
    dst = edge_index[1]

    pad = E_PAD - E
    src_p = jnp.concatenate([src, jnp.zeros((pad,), jnp.int32)])
    dst_p = jnp.concatenate([dst, jnp.full((pad,), TRASH, jnp.int32)])
    tiles = src_p.reshape(NS, NCHUNK, CHUNK)
    gsrc = jnp.stack([tiles, tiles + N])
    dst3 = dst_p.reshape(NS, NCHUNK, CHUNK)

    deg = _sc_deg(dst3)
    degc = deg.reshape(HR * HALF)[:N].reshape(N, 1)

    x2a = h.reshape(N, NC, HALF).transpose(1, 0, 2).reshape(NC * N, HALF)

    agg_a = _sc_segsum(x2a, gsrc, dst3)
    x1, x1_split = _tcb(h, agg_a[:, :N, :], degc, W_self0, W_neigh0, b0)

    x2b = x1_split.reshape(NC * N, HALF)
    agg_b = _sc_segsum(x2b, gsrc, dst3)
    out, _ = _tcb(x1, agg_b[:, :N, :], degc, W_self1, W_neigh1, b1)
    return out


# speedup vs baseline: 1.3100x; 1.3100x over previous
import functools
import jax
import jax.numpy as jnp
from jax import lax
from jax.experimental import pallas as pl
from jax.experimental.pallas import tpu as pltpu
from jax.experimental.pallas import tpu_sc as plsc

N = 10000
E = 160000
D = 256
HALF = 128

NC = 2
NS = 16
CHUNK = 128
NCHUNK = 79
EPT = CHUNK * NCHUNK
E_PAD = EPT * NS
AGG_ROWS = 10240
ROWS_PT = AGG_ROWS // NS
TRASH = N


def _sc_segsum(with_deg):
    DW = HALF
    mesh = plsc.VectorSubcoreMesh(core_axis_name="c", subcore_axis_name="s")

    out_type = [jax.ShapeDtypeStruct((NC, AGG_ROWS, DW), jnp.float32)]
    scratch = [
        pltpu.VMEM((NCHUNK, CHUNK), jnp.int32),
        pltpu.VMEM((NCHUNK, CHUNK), jnp.int32),
        pltpu.VMEM((CHUNK, DW), jnp.float32),
        pltpu.VMEM_SHARED((AGG_ROWS, DW), jnp.float32),
        pltpu.SemaphoreType.DMA,
    ]
    HR = 80
    HPT = 8
    HTILES = HR // HPT
    if with_deg:
        out_type.append(jax.ShapeDtypeStruct((HR, HALF), jnp.float32))
        scratch += [
            pltpu.VMEM((HR, HALF), jnp.float32),
            pltpu.VMEM((8, HALF), jnp.float32),
            pltpu.VMEM((8, HALF), jnp.float32),
        ]

    @functools.partial(
        pl.kernel, mesh=mesh, out_type=tuple(out_type),
        scratch_types=scratch,
        compiler_params=pltpu.CompilerParams(needs_layout_passes=False))
    def k(x2_hbm, gsrc_hbm, dst3_hbm, out_hbm, *rest):
        if with_deg:
            (deg_hbm, idx_v, didx_v, rows_v, agg_sp, sem,
             hist_v, tmp_v, acc_v) = rest
        else:
            idx_v, didx_v, rows_v, agg_sp, sem = rest
        c = lax.axis_index("c")
        s = lax.axis_index("s")
        zero16 = jnp.zeros((16,), jnp.float32)
        ones16 = jnp.ones((16,), jnp.float32)

        pltpu.sync_copy(gsrc_hbm.at[c, s], idx_v)
        pltpu.sync_copy(dst3_hbm.at[s], didx_v)

        def zbody(i, carry):
            for kk in range(DW // 16):
                rows_v[i, pl.ds(kk * 16, 16)] = zero16
            return carry
        lax.fori_loop(0, CHUNK, zbody, 0)
        base = s * ROWS_PT
        for q in range(ROWS_PT // CHUNK):
            pltpu.sync_copy(rows_v, agg_sp.at[pl.ds(base + q * CHUNK, CHUNK)])
        if with_deg:
            def hzbody(i, carry):
                for kk in range(HALF // 16):
                    hist_v[i, pl.ds(kk * 16, 16)] = zero16
                return carry
            lax.fori_loop(0, HR, hzbody, 0)
        plsc.subcore_barrier()

        def body(j, carry):
            pltpu.async_copy(x2_hbm.at[idx_v.at[j]], rows_v, sem).wait()
            pltpu.sync_copy(rows_v, agg_sp.at[didx_v.at[j]], add=True)
            if with_deg:
                @pl.when(c == 0)
                def _():
                    for kk in range(CHUNK // 16):
                        d16 = didx_v[j, pl.ds(kk * 16, 16)]
                        plsc.addupdate_scatter(
                            hist_v,
                            [lax.shift_right_logical(d16, 7),
                             lax.bitwise_and(d16, 127)],
                            ones16)
            return carry
        lax.fori_loop(0, NCHUNK, body, 0)
        plsc.subcore_barrier()

        pltpu.sync_copy(agg_sp.at[pl.ds(base, ROWS_PT)],
                        out_hbm.at[c, pl.ds(base, ROWS_PT)])

        if with_deg:
            plsc.subcore_barrier()

            @pl.when(c == 0)
            def _():
                pltpu.sync_copy(hist_v, agg_sp.at[pl.ds(s * HR, HR)])
            plsc.subcore_barrier()

            @pl.when(jnp.logical_and(c == 0, s < HTILES))
            def _():
                def azbody(i, carry):
                    acc_v[i // 8, pl.ds((i % 8) * 16, 16)] = zero16
                    return carry
                lax.fori_loop(0, HPT * (HALF // 16), azbody, 0)

                def tbody(t, carry):
                    off = pl.multiple_of(t * HR + s * HPT, 8)
                    pltpu.sync_copy(agg_sp.at[pl.ds(off, HPT)], tmp_v)

                    def abody(i, carry2):
                        rr = i // 8
                        sl = pl.ds((i % 8) * 16, 16)
                        acc_v[rr, sl] = acc_v[rr, sl] + tmp_v[rr, sl]
                        return carry2
                    return lax.fori_loop(0, HPT * (HALF // 16), abody, carry)
                lax.fori_loop(0, NS, tbody, 0)
                pltpu.sync_copy(
                    acc_v,
                    deg_hbm.at[pl.ds(pl.multiple_of(s * HPT, 8), HPT)])

    return k


_sc_segsum_deg = _sc_segsum(True)
_sc_segsum_plain = _sc_segsum(False)

_BLK = 1000


def _tc_layer_body(x_ref, agg_ref, deg_ref, ws_ref, wn_ref, b_ref,
                   y_ref, y2_ref):
    r = 1.0 / jnp.maximum(deg_ref[...], 1.0)
    lo = agg_ref[0] * r
    hi = agg_ref[1] * r
    acc = jnp.dot(x_ref[...], ws_ref[...], preferred_element_type=jnp.float32)
    acc += jnp.dot(lo, wn_ref[0:HALF, :], preferred_element_type=jnp.float32)
    acc += jnp.dot(hi, wn_ref[HALF:, :], preferred_element_type=jnp.float32)
    y = jnp.maximum(acc + b_ref[...], 0.0)
    y_ref[...] = y
    y2_ref[0] = y[:, :HALF]
    y2_ref[1] = y[:, HALF:]


def _tc_layer(x, agg2, degc, w_self, w_neigh, b):
    return pl.pallas_call(
        _tc_layer_body,
        grid=(N // _BLK,),
        in_specs=[
            pl.BlockSpec((_BLK, D), lambda i: (i, 0)),
            pl.BlockSpec((NC, _BLK, HALF), lambda i: (0, i, 0)),
            pl.BlockSpec((_BLK, 1), lambda i: (i, 0)),
            pl.BlockSpec((D, D), lambda i: (0, 0)),
            pl.BlockSpec((D, D), lambda i: (0, 0)),
            pl.BlockSpec((1, D), lambda i: (0, 0)),
        ],
        out_specs=[
            pl.BlockSpec((_BLK, D), lambda i: (i, 0)),
            pl.BlockSpec((NC, _BLK, HALF), lambda i: (0, i, 0)),
        ],
        out_shape=[
            jax.ShapeDtypeStruct((N, D), jnp.float32),
            jax.ShapeDtypeStruct((NC, N, HALF), jnp.float32),
        ],
    )(x, agg2, degc, w_self, w_neigh, b.reshape(1, D))


def kernel(h, edge_index, W_self0, W_neigh0, b0, W_self1, W_neigh1, b1):
    src = edge_index[0]
    dst = edge_index[1]

    pad = E_PAD - E
    src_p = jnp.concatenate([src, jnp.zeros((pad,), jnp.int32)])
    dst_p = jnp.concatenate([dst, jnp.full((pad,), TRASH, jnp.int32)])
    tiles = src_p.reshape(NS, NCHUNK, CHUNK)
    gsrc = jnp.stack([tiles, tiles + N])
    dst3 = dst_p.reshape(NS, NCHUNK, CHUNK)

    x2a = h.reshape(N, NC, HALF).transpose(1, 0, 2).reshape(NC * N, HALF)

    agg_a, deg = _sc_segsum_deg(x2a, gsrc, dst3)
    degc = deg.reshape(AGG_ROWS)[:N].reshape(N, 1)
    agg0 = agg_a[:, :N, :]

    x1, x1_split = _tc_layer(h, agg0, degc, W_self0, W_neigh0, b0)

    x2b = x1_split.reshape(NC * N, HALF)
    (agg_b,) = _sc_segsum_plain(x2b, gsrc, dst3)
    agg1 = agg_b[:, :N, :]

    out, _ = _tc_layer(x1, agg1, degc, W_self1, W_neigh1, b1)
    return out
